# baseline (speedup 1.0000x reference)
import functools

import jax
import jax.numpy as jnp
from jax.experimental import pallas as pl

N = 10000
CB = 2048
RB = 400
MB = 2000
NCB = 5
NRB = 25
NPAD = NCB * CB


def _deg_kernel(a_ref, deg_ref):
    rb = pl.program_id(1)

    @pl.when(rb == 0)
    def _():
        deg_ref[...] = jnp.zeros_like(deg_ref)

    deg_ref[...] += jnp.sum(a_ref[...], axis=0, keepdims=True)


def _mm_kernel(h_ref, w_ref, deg_ref, y_ref, *, act):
    h = h_ref[...]
    if act:
        h = jnp.where(h > 0, h, jnp.exp(h) - 1.0)
    y = jax.lax.dot_general(h, w_ref[...], (((1,), (0,)), ((), ())),
                            preferred_element_type=jnp.float32)
    s = jax.lax.rsqrt(deg_ref[...] + 1.0)
    y_ref[...] = s * y


def _layer_kernel(a_ref, y_ref, ycol_ref, deg_ref, b_ref, out_ref):
    rb = pl.program_id(1)

    @pl.when(rb == 0)
    def _():
        out_ref[...] = jnp.zeros_like(out_ref)

    out_ref[...] += jax.lax.dot_general(
        a_ref[...], y_ref[...], (((0,), (0,)), ((), ())),
        preferred_element_type=jnp.float32)

    @pl.when(rb == NRB - 1)
    def _():
        s = jax.lax.rsqrt(deg_ref[...] + 1.0)
        out_ref[...] = s * (out_ref[...] + ycol_ref[...]) + b_ref[...]


def _degrees(a):
    return pl.pallas_call(
        _deg_kernel,
        grid=(NCB, NRB),
        in_specs=[pl.BlockSpec((RB, CB), lambda cb, rb: (rb, cb))],
        out_specs=pl.BlockSpec((1, CB), lambda cb, rb: (0, cb)),
        out_shape=jax.ShapeDtypeStruct((1, NPAD), jnp.float32),
    )(a)


def _feature_mm(h, w, deg_col, act):
    m, d_out = h.shape[0], w.shape[1]
    return pl.pallas_call(
        functools.partial(_mm_kernel, act=act),
        grid=(m // MB,),
        in_specs=[
            pl.BlockSpec((MB, h.shape[1]), lambda i: (i, 0)),
            pl.BlockSpec(w.shape, lambda i: (0, 0)),
            pl.BlockSpec((MB, 1), lambda i: (i, 0)),
        ],
        out_specs=pl.BlockSpec((MB, d_out), lambda i: (i, 0)),
        out_shape=jax.ShapeDtypeStruct((m, d_out), jnp.float32),
    )(h, w, deg_col)


def _aggregate(a, y, deg_col, b):
    d = y.shape[1]
    return pl.pallas_call(
        _layer_kernel,
        grid=(NCB, NRB),
        in_specs=[
            pl.BlockSpec((RB, CB), lambda cb, rb: (rb, cb)),
            pl.BlockSpec((RB, d), lambda cb, rb: (rb, 0)),
            pl.BlockSpec((CB, d), lambda cb, rb: (cb, 0)),
            pl.BlockSpec((CB, 1), lambda cb, rb: (cb, 0)),
            pl.BlockSpec((1, d), lambda cb, rb: (0, 0)),
        ],
        out_specs=pl.BlockSpec((CB, d), lambda cb, rb: (cb, 0)),
        out_shape=jax.ShapeDtypeStruct((NPAD, d), jnp.float32),
    )(a, y, y, deg_col, b.reshape(1, d))


def kernel(x, adj_matrix, W1, b1, W2, b2):
    a = adj_matrix[0]
    xm = x[0]
    deg = _degrees(a)
    deg_col = deg.reshape(NPAD, 1)
    y1 = _feature_mm(xm, W1, deg_col[:N], act=False)
    h1 = _aggregate(a, y1, deg_col, b1)
    y2 = _feature_mm(h1[:N], W2, deg_col[:N], act=True)
    out = _aggregate(a, y2, deg_col, b2)
    return out[:N].reshape(1, N, 32)
